# baseline (speedup 1.0000x reference)
import functools

import jax
import jax.numpy as jnp
from jax import lax
from jax.experimental import pallas as pl
from jax.experimental.pallas import tpu as pltpu
from jax.experimental.pallas import tpu_sc as plsc

NC = 2
NS = 16
NW = NC * NS
L = 16

B = 16384
D = 64
BW = B // NW
CH = 4
CW = BW // CH


def _rsqrt_nr(x):
    i = lax.bitcast_convert_type(x, jnp.int32)
    y = lax.bitcast_convert_type(jnp.int32(0x5F3759DF) - (i >> 1), jnp.float32)
    half_x = 0.5 * x
    for _ in range(3):
        y = y * (1.5 - half_x * y * y)
    return y


def _pow34(x):
    s = x * _rsqrt_nr(x)
    q = s * _rsqrt_nr(s)
    return s * q


def _k1_body(wi_hbm, wj_hbm, wt_hbm, wc_hbm, yp_hbm, pw_hbm,
             idxi_v, idxj_v, ei_v, ej_v, yp_v, pw_v, sem):
    wid = lax.axis_index("s") * NC + lax.axis_index("c")

    pltpu.sync_copy(wi_hbm.at[wid], idxi_v)
    pltpu.sync_copy(wj_hbm.at[wid], idxj_v)

    copies = []
    for j in range(CH):
        copies.append(pltpu.async_copy(
            wt_hbm.at[idxi_v.at[j]], ei_v.at[pl.ds(j * CW, CW)], sem))
        copies.append(pltpu.async_copy(
            wc_hbm.at[idxj_v.at[j]], ej_v.at[pl.ds(j * CW, CW)], sem))
    for c in copies:
        c.wait()

    lane = lax.iota(jnp.int32, L)

    def group_body(g, carry):
        rows = g * L + lane
        acc = jnp.zeros((L,), jnp.float32)
        for d in range(D):
            cols = jnp.full((L,), d, jnp.int32)
            a = plsc.load_gather(ei_v, [rows, cols])
            b = plsc.load_gather(ej_v, [rows, cols])
            acc = acc + a * b
        yp_v[pl.ds(g * L, L)] = acc
        return carry

    lax.fori_loop(0, BW // L, group_body, jnp.int32(0))

    pltpu.sync_copy(yp_v, yp_hbm.at[pl.ds(wid * BW, BW)])

    def pow_body(v, acc):
        x = yp_v[pl.ds(v * L, L)] / jnp.float32(100.0)
        return acc + _pow34(x)

    acc = lax.fori_loop(0, BW // L, pow_body, jnp.zeros((L,), jnp.float32))
    pw_v[...] = acc
    pltpu.sync_copy(pw_v, pw_hbm.at[wid])


@functools.lru_cache(maxsize=1)
def _get_k1():
    return pl.kernel(
        _k1_body,
        out_type=[
            jax.ShapeDtypeStruct((B,), jnp.float32),
            jax.ShapeDtypeStruct((NW, L), jnp.float32),
        ],
        mesh=plsc.VectorSubcoreMesh(core_axis_name="c", subcore_axis_name="s"),
        compiler_params=pltpu.CompilerParams(
            needs_layout_passes=False, use_tc_tiling_on_sc=False),
        scratch_types=[
            pltpu.VMEM((CH, CW), jnp.int32),
            pltpu.VMEM((CH, CW), jnp.int32),
            pltpu.VMEM((BW, D), jnp.float32),
            pltpu.VMEM((BW, D), jnp.float32),
            pltpu.VMEM((BW,), jnp.float32),
            pltpu.VMEM((L,), jnp.float32),
            pltpu.SemaphoreType.DMA,
        ],
    )


def _k2_body(yp_ref, yt_ref, pw_ref, o_ref):
    ws = jnp.sum(pw_ref[...])
    d = yp_ref[...] - jnp.log(yt_ref[...].astype(jnp.float32))
    o_ref[...] = ws * (d * d)


def kernel(w_i, w_j, y_true, W_target, W_context):
    wi3 = w_i.reshape(NW, CH, CW)
    wj3 = w_j.reshape(NW, CH, CW)
    ypred, partials = _get_k1()(wi3, wj3, W_target, W_context)
    out2d = pl.pallas_call(
        _k2_body,
        out_shape=jax.ShapeDtypeStruct((128, 128), jnp.float32),
    )(ypred.reshape(128, 128), y_true.reshape(128, 128), partials)
    return out2d.reshape(B, 1)
